# baseline (device time: 665557 ns/iter reference)
import jax
import jax.numpy as jnp
from jax import lax
from jax.experimental import pallas as pl
from jax.experimental.pallas import tpu as pltpu

M = 4096
HALF = M // 2
NC = 4
R = HALF // NC
S = 256
NSUB = 2 * NC
KB = 256
K = 8192
NK = K // KB
MESH = pl.DeviceIdType.MESH


def kernel(dy, W):
    n = W.shape[0]

    def body(dy_hbm, w_hbm, out_ref, w_buf, dy_buf, acc, yrecv,
             s1_send, s1_recv, s2_send, s2_recv, w_sem, dy_sem, out_sem):
        my_x = lax.axis_index("x")
        my_y = lax.axis_index("y")
        row0 = my_x * HALF

        def load(c, k, slot):
            w_cp = pltpu.make_async_copy(
                w_hbm.at[:, pl.ds(k * KB, KB)], w_buf.at[slot],
                w_sem.at[slot])
            dy_cp = pltpu.make_async_copy(
                dy_hbm.at[pl.ds(row0 + c * R, R), pl.ds(k * KB, KB)],
                dy_buf.at[slot], dy_sem.at[slot])
            return w_cp, dy_cp

        def acc_sub(j):
            return acc.at[(j // 2) % 2].at[pl.ds((j % 2) * S, S), :]

        def rdma1(j):
            return pltpu.make_async_remote_copy(
                src_ref=acc_sub(j), dst_ref=yrecv.at[j % 4],
                send_sem=s1_send.at[j], recv_sem=s1_recv.at[j],
                device_id=(my_x, 1 - my_y), device_id_type=MESH)

        def cp_out(j):
            return pltpu.make_async_copy(
                acc_sub(j),
                out_ref.at[pl.ds(row0 + j * S, S), :], out_sem.at[j])

        def rdma2(j):
            return pltpu.make_async_remote_copy(
                src_ref=acc_sub(j),
                dst_ref=out_ref.at[pl.ds(row0 + j * S, S), :],
                send_sem=s2_send.at[j], recv_sem=s2_recv.at[j],
                device_id=(1 - my_x, my_y), device_id_type=MESH)

        def process_comm(j):
            rdma1(j).wait_send()
            rdma1(j).wait_recv()
            sl = (j // 2) % 2
            rows = pl.ds((j % 2) * S, S)
            acc[sl, rows, :] = acc[sl, rows, :] + yrecv[j % 4, :, :]
            cp_out(j).start()
            rdma2(j).start()

        barrier = pltpu.get_barrier_semaphore()
        pl.semaphore_signal(barrier, inc=1, device_id=(my_x, 1 - my_y),
                            device_id_type=MESH)
        pl.semaphore_signal(barrier, inc=1, device_id=(1 - my_x, my_y),
                            device_id_type=MESH)
        pl.semaphore_wait(barrier, 2)

        for cp in load(0, 0, 0):
            cp.start()
        for c in range(NC):
            if c >= 2:
                for j in (2 * c - 4, 2 * c - 3):
                    cp_out(j).wait()
                    rdma2(j).wait_send()
            for cp in load(c, 1, 1):
                cp.start()
            for cp in load(c, 0, 0):
                cp.wait()
            acc[c % 2, :, :] = lax.dot_general(
                dy_buf[0], w_buf[0], (((1,), (1,)), ((), ())),
                preferred_element_type=jnp.float32)

            def k_body(k, _, c=c):
                slot = lax.rem(k, 2)
                nxt = lax.rem(k + 1, 2)

                @pl.when(k + 1 < NK)
                def _():
                    for cp in load(c, k + 1, nxt):
                        cp.start()

                for cp in load(c, k, slot):
                    cp.wait()
                acc[c % 2, :, :] += lax.dot_general(
                    dy_buf[slot], w_buf[slot], (((1,), (1,)), ((), ())),
                    preferred_element_type=jnp.float32)
                return 0

            lax.fori_loop(1, NK, k_body, 0)
            if c + 1 < NC:
                for cp in load(c + 1, 0, 0):
                    cp.start()
            if c >= 1:
                process_comm(2 * c - 2)
            rdma1(2 * c).start()
            if c >= 1:
                process_comm(2 * c - 1)
            rdma1(2 * c + 1).start()
        process_comm(NSUB - 2)
        process_comm(NSUB - 1)

        for j in range(NSUB - 4, NSUB):
            cp_out(j).wait()
            rdma2(j).wait_send()
        for j in range(NSUB):
            rdma2(j).wait_recv()

    return pl.pallas_call(
        body,
        out_shape=jax.ShapeDtypeStruct((M, n), jnp.float32),
        in_specs=[pl.BlockSpec(memory_space=pltpu.MemorySpace.HBM),
                  pl.BlockSpec(memory_space=pltpu.MemorySpace.HBM)],
        out_specs=pl.BlockSpec(memory_space=pltpu.MemorySpace.HBM),
        scratch_shapes=[
            pltpu.VMEM((2, n, KB), jnp.float32),
            pltpu.VMEM((2, R, KB), jnp.float32),
            pltpu.VMEM((2, R, n), jnp.float32),
            pltpu.VMEM((4, S, n), jnp.float32),
            pltpu.SemaphoreType.DMA((NSUB,)),
            pltpu.SemaphoreType.DMA((NSUB,)),
            pltpu.SemaphoreType.DMA((NSUB,)),
            pltpu.SemaphoreType.DMA((NSUB,)),
            pltpu.SemaphoreType.DMA((2,)),
            pltpu.SemaphoreType.DMA((2,)),
            pltpu.SemaphoreType.DMA((NSUB,)),
        ],
        compiler_params=pltpu.CompilerParams(
            collective_id=0, vmem_limit_bytes=60 * 1024 * 1024),
    )(dy, W)
